# Initial kernel scaffold; baseline (speedup 1.0000x reference)
import jax
import jax.numpy as jnp
from jax.experimental import pallas as pl


def kernel(base_feat, im_info, gt_boxes, num_boxes, W_conv, b_conv, W_cls, b_cls, W_bbox, b_bbox):
    raise NotImplementedError("write your pallas kernel here")


# speedup vs baseline: 102.2053x; 102.2053x over previous
import functools

import jax
import jax.numpy as jnp
import numpy as np
from jax import lax
from jax.experimental import pallas as pl
from jax.experimental.pallas import tpu as pltpu

_FEAT_STRIDE = 16
_PRE_NMS = 6000
_POST_NMS = 300
_NMS_THRESH = 0.7
_A = 9


def _whctrs(anchor):
    w = anchor[2] - anchor[0] + 1.0
    h = anchor[3] - anchor[1] + 1.0
    return w, h, anchor[0] + 0.5 * (w - 1), anchor[1] + 0.5 * (h - 1)


def _mkanch(ws, hs, x_ctr, y_ctr):
    ws = ws[:, None]
    hs = hs[:, None]
    return np.hstack((x_ctr - 0.5 * (ws - 1), y_ctr - 0.5 * (hs - 1),
                      x_ctr + 0.5 * (ws - 1), y_ctr + 0.5 * (hs - 1)))


def _base_anchors():
    base = np.array([1.0, 1.0, 16.0, 16.0]) - 1.0
    ratios = np.array([0.5, 1.0, 2.0])
    scales = np.array([8.0, 16.0, 32.0])
    w, h, xc, yc = _whctrs(base)
    size = w * h
    ws = np.round(np.sqrt(size / ratios))
    hs = np.round(ws * ratios)
    ratio_anchors = _mkanch(ws, hs, xc, yc)
    outs = []
    for i in range(ratio_anchors.shape[0]):
        w, h, xc, yc = _whctrs(ratio_anchors[i, :])
        outs.append(_mkanch(w * scales, h * scales, xc, yc))
    return np.vstack(outs).astype(np.float32)


@functools.lru_cache(maxsize=4)
def _anchor_aux(H, W):
    Wp = ((W + 2 + 7) // 8) * 8
    MP = ((H * Wp + 127) // 128) * 128
    anch = _base_anchors()
    m = np.arange(MP)
    y = (m // Wp).astype(np.float32) * _FEAT_STRIDE
    x = (m % Wp).astype(np.float32) * _FEAT_STRIDE
    ax1 = (anch[None, :, 0] + x[:, None]).astype(np.float32)
    ay1 = (anch[None, :, 1] + y[:, None]).astype(np.float32)
    ax2 = (anch[None, :, 2] + x[:, None]).astype(np.float32)
    ay2 = (anch[None, :, 3] + y[:, None]).astype(np.float32)
    wa = ax2 - ax1 + 1.0
    ha = ay2 - ay1 + 1.0
    cxa = ax1 + 0.5 * wa
    cya = ay1 + 0.5 * ha
    return wa, ha, cxa, cya


def _rpn_head_body(H, W, MO, xt_ref, w9_ref, bconv_ref, whead_ref, bhead_ref,
                   aux_ref, iminfo_ref, cls_ref, boxes_ref, acc_ref):
    Wp = ((W + 2 + 7) // 8) * 8
    t = pl.program_id(1)
    off = pl.multiple_of((t // 3) * Wp, 8)
    xs = xt_ref[0, 0, pl.ds(off, MO), :]
    d = jnp.dot(xs, w9_ref[0], preferred_element_type=jnp.float32)

    @pl.when(t == 0)
    def _():
        acc_ref[...] = d

    @pl.when(t > 0)
    def _():
        acc_ref[...] += d

    @pl.when(t == pl.num_programs(1) - 1)
    def _():
        a = jnp.maximum(acc_ref[...] + bconv_ref[...], 0.0)
        h = jnp.dot(a, whead_ref[...],
                    preferred_element_type=jnp.float32) + bhead_ref[...]
        cls_ref[0] = h[:, 0:18]

        dx = h[:, 18:27]
        dy = h[:, 27:36]
        dw = h[:, 36:45]
        dh = h[:, 45:54]
        wa = aux_ref[:, 0:9]
        ha = aux_ref[:, 9:18]
        pcx = dx * wa + aux_ref[:, 18:27]
        pcy = dy * ha + aux_ref[:, 27:36]
        pw = jnp.exp(dw) * wa
        ph = jnp.exp(dh) * ha
        imh = iminfo_ref[0, 0, 0]
        imw = iminfo_ref[0, 0, 1]
        x1 = jnp.clip(pcx - 0.5 * pw, 0.0, imw - 1.0)
        y1 = jnp.clip(pcy - 0.5 * ph, 0.0, imh - 1.0)
        x2 = jnp.clip(pcx + 0.5 * pw, 0.0, imw - 1.0)
        y2 = jnp.clip(pcy + 0.5 * ph, 0.0, imh - 1.0)
        boxes_ref[0] = jnp.concatenate([x1, y1, x2, y2], axis=1)


def _rpn_head(base_feat, W_conv, b_conv, W_cls, b_cls, W_bbox, b_bbox, im_info):
    B, C, H, W = base_feat.shape
    Wp = ((W + 2 + 7) // 8) * 8
    M = H * Wp
    MO = ((M + 127) // 128) * 128
    MX = MO + 128
    CO = W_conv.shape[0]

    xpad = jnp.pad(base_feat, ((0, 0), (0, 0), (1, 1), (1, Wp - W - 1)))
    xt = xpad.reshape(B, C, (H + 2) * Wp).transpose(0, 2, 1)
    xt = jnp.pad(xt, ((0, 0), (0, MX + 2 - (H + 2) * Wp), (0, 0)))
    xt = jnp.stack([xt[:, j:j + MX, :] for j in range(3)], axis=1)

    w9 = jnp.stack([W_conv[:, :, t // 3, t % 3].T for t in range(9)], axis=0)

    wcls = W_cls[:, :, 0, 0]
    wbb = W_bbox[:, :, 0, 0]
    cols = [wcls[0:9], wcls[9:18],
            wbb[0::4], wbb[1::4], wbb[2::4], wbb[3::4]]
    whead = jnp.concatenate(cols, axis=0)
    whead = jnp.pad(whead, ((0, 10), (0, 0))).T
    bhead = jnp.concatenate([b_cls[0:9], b_cls[9:18], b_bbox[0::4],
                             b_bbox[1::4], b_bbox[2::4], b_bbox[3::4]])
    bhead = jnp.pad(bhead, (0, 10)).reshape(1, 64)

    wa, ha, cxa, cya = _anchor_aux(H, W)
    aux = jnp.asarray(np.concatenate(
        [wa[:MO], ha[:MO], cxa[:MO], cya[:MO]], axis=1))

    grid = (B, 9)
    cls, boxes = pl.pallas_call(
        functools.partial(_rpn_head_body, H, W, MO),
        grid=grid,
        in_specs=[
            pl.BlockSpec((1, 1, MX, C), lambda b, t: (b, t % 3, 0, 0)),
            pl.BlockSpec((1, C, CO), lambda b, t: (t, 0, 0)),
            pl.BlockSpec((1, CO), lambda b, t: (0, 0)),
            pl.BlockSpec((CO, 64), lambda b, t: (0, 0)),
            pl.BlockSpec((1, 64), lambda b, t: (0, 0)),
            pl.BlockSpec((MO, 4 * _A), lambda b, t: (0, 0)),
            pl.BlockSpec((1, 1, 3), lambda b, t: (b, 0, 0)),
        ],
        out_specs=[
            pl.BlockSpec((1, MO, 2 * _A), lambda b, t: (b, 0, 0)),
            pl.BlockSpec((1, MO, 4 * _A), lambda b, t: (b, 0, 0)),
        ],
        out_shape=[
            jax.ShapeDtypeStruct((B, MO, 2 * _A), jnp.float32),
            jax.ShapeDtypeStruct((B, MO, 4 * _A), jnp.float32),
        ],
        scratch_shapes=[pltpu.VMEM((MO, CO), jnp.float32)],
    )(xt, w9, b_conv.reshape(1, CO), whead, bhead, aux,
      im_info.reshape(B, 1, 3))
    return cls, boxes


def _nms_body(NB, n_valid, x1r_ref, y1r_ref, x2r_ref, y2r_ref,
              x1c_ref, y1c_ref, x2c_ref, y2c_ref, keep_ref, sup_ref):
    N = NB * 128
    thresh = _NMS_THRESH

    sup_ref[...] = jnp.zeros_like(sup_ref)

    cx1 = x1c_ref[0]
    cy1 = y1c_ref[0]
    cx2 = x2c_ref[0]
    cy2 = y2c_ref[0]
    areas_c = (cx2 - cx1 + 1.0) * (cy2 - cy1 + 1.0)
    flat_col = (lax.broadcasted_iota(jnp.int32, (NB, 128), 0) * 128
                + lax.broadcasted_iota(jnp.int32, (NB, 128), 1))

    def block_step(i, _):
        sl = pl.ds(i * 128, 128)
        rx1 = x1r_ref[0, sl, 0]
        ry1 = y1r_ref[0, sl, 0]
        rx2 = x2r_ref[0, sl, 0]
        ry2 = y2r_ref[0, sl, 0]
        ar = ((rx2 - rx1 + 1.0) * (ry2 - ry1 + 1.0))

        ox1 = x1c_ref[0, i][None, :]
        oy1 = y1c_ref[0, i][None, :]
        ox2 = x2c_ref[0, i][None, :]
        oy2 = y2c_ref[0, i][None, :]
        oar = (ox2 - ox1 + 1.0) * (oy2 - oy1 + 1.0)
        xx1 = jnp.maximum(rx1[:, None], ox1)
        yy1 = jnp.maximum(ry1[:, None], oy1)
        xx2 = jnp.minimum(rx2[:, None], ox2)
        yy2 = jnp.minimum(ry2[:, None], oy2)
        w = jnp.maximum(0.0, xx2 - xx1 + 1.0)
        h = jnp.maximum(0.0, yy2 - yy1 + 1.0)
        inter = w * h
        iou = inter / (ar[:, None] + oar - inter)
        ridx = lax.broadcasted_iota(jnp.int32, (128, 128), 0)
        cidx = lax.broadcasted_iota(jnp.int32, (128, 128), 1)
        S = jnp.where((iou > thresh) & (ridx < cidx), 1.0, 0.0)

        base = i * 128
        lidx = lax.broadcasted_iota(jnp.int32, (1, 128), 1) + base
        presup = sup_ref[pl.ds(i, 1), :]
        k0 = jnp.where((lidx < n_valid) & (presup == 0.0), 1.0, 0.0)

        def fcond(c):
            return c[1]

        def fbody(c):
            kv, _ = c
            s = jnp.dot(kv, S, preferred_element_type=jnp.float32)
            knew = jnp.where(s > 0.0, 0.0, k0)
            return knew, jnp.any(knew != kv)

        kfin, _ = lax.while_loop(fcond, fbody, (k0, True))
        keep_ref[0, pl.ds(i, 1), :] = kfin

        xx1 = jnp.maximum(rx1[:, None, None], cx1[None])
        yy1 = jnp.maximum(ry1[:, None, None], cy1[None])
        xx2 = jnp.minimum(rx2[:, None, None], cx2[None])
        yy2 = jnp.minimum(ry2[:, None, None], cy2[None])
        w = jnp.maximum(0.0, xx2 - xx1 + 1.0)
        h = jnp.maximum(0.0, yy2 - yy1 + 1.0)
        inter = w * h
        iou = inter / (ar[:, None, None] + areas_c[None] - inter)
        m2 = jnp.where(iou > thresh, 1.0, 0.0).reshape(128, N)
        contrib = jnp.dot(kfin, m2, preferred_element_type=jnp.float32)
        contrib = contrib.reshape(NB, 128)
        newsup = jnp.where((contrib > 0.0) & (flat_col >= base + 128),
                           1.0, 0.0)
        sup_ref[...] = jnp.maximum(sup_ref[...], newsup)
        return 0

    lax.fori_loop(0, NB, block_step, 0)


def _nms(props_top):
    B, N, _ = props_top.shape
    NB = N // 128
    x1r = props_top[:, :, 0:1]
    y1r = props_top[:, :, 1:2]
    x2r = props_top[:, :, 2:3]
    y2r = props_top[:, :, 3:4]
    cview = props_top.reshape(B, NB, 128, 4)
    x1c = cview[..., 0]
    y1c = cview[..., 1]
    x2c = cview[..., 2]
    y2c = cview[..., 3]
    keep = pl.pallas_call(
        functools.partial(_nms_body, NB, _PRE_NMS),
        grid=(B,),
        in_specs=[pl.BlockSpec((1, N, 1), lambda b: (b, 0, 0))] * 4
        + [pl.BlockSpec((1, NB, 128), lambda b: (b, 0, 0))] * 4,
        out_specs=pl.BlockSpec((1, NB, 128), lambda b: (b, 0, 0)),
        out_shape=jax.ShapeDtypeStruct((B, NB, 128), jnp.float32),
        scratch_shapes=[pltpu.VMEM((NB, 128), jnp.float32)],
    )(x1r, y1r, x2r, y2r, x1c, y1c, x2c, y2c)
    return keep.reshape(B, N)


def kernel(base_feat, im_info, gt_boxes, num_boxes, W_conv, b_conv,
           W_cls, b_cls, W_bbox, b_bbox):
    B, C, H, W = base_feat.shape
    cls, boxes = _rpn_head(base_feat, W_conv, b_conv, W_cls, b_cls,
                           W_bbox, b_bbox, im_info)
    MO = cls.shape[1]
    bg = cls[..., 0:9]
    fg = cls[..., 9:18]
    m = jnp.maximum(bg, fg)
    ebg = jnp.exp(bg - m)
    efg = jnp.exp(fg - m)
    p = efg / (ebg + efg)
    Wp = ((W + 2 + 7) // 8) * 8
    mi = lax.broadcasted_iota(jnp.int32, (1, MO, _A), 1)
    valid = ((mi % Wp) < W) & (mi < H * Wp)
    scores = jnp.where(valid, p, -1.0).reshape(B, MO * _A)
    boxes4 = boxes.reshape(B, MO, 4, _A).transpose(0, 1, 3, 2).reshape(
        B, MO * _A, 4)

    _, order = lax.top_k(scores, _PRE_NMS)
    props_top = jnp.take_along_axis(boxes4, order[:, :, None], axis=1)
    NPAD = ((_PRE_NMS + 127) // 128) * 128
    props_pad = jnp.pad(props_top, ((0, 0), (0, NPAD - _PRE_NMS), (0, 0)))

    keep = _nms(props_pad)

    rois = []
    for b in range(B):
        keep_idx = jnp.nonzero(keep[b] > 0.5, size=_POST_NMS, fill_value=0)[0]
        props_keep = props_top[b][keep_idx]
        batch_col = jnp.full((_POST_NMS, 1), float(b), dtype=props_keep.dtype)
        rois.append(jnp.concatenate([batch_col, props_keep], axis=1))
    return jnp.stack(rois, axis=0)
